# baseline (speedup 1.0000x reference)
import jax
import jax.numpy as jnp
from jax import lax
from jax.experimental import pallas as pl
from jax.experimental.pallas import tpu as pltpu
from jax.experimental.pallas import tpu_sc as plsc

_NW = 32
_W = 128
_ROWS = 125
_VPR = _W // 16
_MAXK = 8

_PHI_N = 200
_PHI2_N = 100
_PHI_PAD = 208
_PHI2_PAD = 112
_HIDDEN = 3
_F1G = _PHI_N * 32 - 32
_F2G = (_PHI2_N - 1) * 64
_F1PAD = 6384
_F2PAD = 6352


def _sc_body(*refs, assemble):
    if assemble:
        (x1_hbm, x2_hbm, phi_hbm, big_hbm, par_hbm, prev_hbm, out_hbm,
         xb0, xb1, ob0, ob1, phib, bigb, parb, f1b, f2b,
         isem0, isem1, osem0, osem1) = refs
        base0 = prev_hbm.shape[0]
    else:
        (x1_hbm, x2_hbm, phi_hbm, big_hbm, par_hbm, out_hbm,
         xb0, xb1, ob0, ob1, phib, bigb, parb, f1b, f2b,
         isem0, isem1, osem0, osem1) = refs
        base0 = 0
    wid = lax.axis_index("s") * 2 + lax.axis_index("c")

    pltpu.sync_copy(phi_hbm, phib)
    pltpu.sync_copy(big_hbm, bigb)
    pltpu.sync_copy(par_hbm, parb)
    eta_v = parb[pl.ds(0, 16)]
    lam_v = parb[pl.ds(16, 16)]
    shift = [eta_v * (float(_F1G) * q) + 0.5 for q in range(_HIDDEN)]
    lam2 = lam_v * (float(_F2G) / 6.0)
    cst2 = [(q + 3.0) * (float(_F2G) / 6.0) + 0.5 for q in range(_HIDDEN)]

    iota = lax.iota(jnp.int32, 16)

    @plsc.parallel_loop(0, _F1PAD // 16)
    def _build1(j):
        g = iota + j * 16
        u = jnp.minimum(g.astype(jnp.float32) * (1.0 / 32.0),
                        float(_PHI_N - 1))
        ii = u.astype(jnp.int32)
        t = u - ii.astype(jnp.float32)
        c0 = plsc.load_gather(phib, [ii])
        c1 = plsc.load_gather(phib, [ii + 1])
        f1b[pl.ds(j * 16, 16)] = c0 + t * (c1 - c0)

    @plsc.parallel_loop(0, _F2PAD // 16)
    def _build2(j):
        g = iota + j * 16
        u = jnp.minimum(g.astype(jnp.float32) * (1.0 / 64.0),
                        float(_PHI2_N - 1))
        ii = u.astype(jnp.int32)
        t = u - ii.astype(jnp.float32)
        d0 = plsc.load_gather(bigb, [ii])
        d1 = plsc.load_gather(bigb, [ii + 1])
        f2b[pl.ds(j * 16, 16)] = d0 + t * (d1 - d0)

    xbufs, obufs = (xb0, xb1), (ob0, ob1)
    isems, osems = (isem0, isem1), (osem0, osem1)

    def compute(b):
        xb, ob = xbufs[b], obufs[b]

        @plsc.parallel_loop(0, _ROWS)
        def _row(r):
            for c in range(_VPR):
                v = xb[r, pl.ds(c * 16, 16)]
                acc = None
                for q in range(_HIDDEN):
                    g1 = jnp.minimum(v * float(_F1G) + shift[q], _F1G + 0.49)
                    phi = plsc.load_gather(f1b, [g1.astype(jnp.int32)])
                    g2 = jnp.clip(phi * lam2 + cst2[q], 0.0, _F2G + 0.49)
                    r_ = plsc.load_gather(f2b, [g2.astype(jnp.int32)])
                    acc = r_ if acc is None else acc + r_
                ob[r, pl.ds(c * 16, 16)] = acc

    def run_pipeline(x_hbm, widl, nw, out_base):
        nchunks = x_hbm.shape[0] // _ROWS
        maxk = -(-nchunks // nw)
        assert maxk % 2 == 0 and maxk >= 4
        nk = jnp.where(widl < nchunks % nw, maxk, maxk - 1)

        def start_in(k, b):
            off = (widl + nw * k) * _ROWS
            pltpu.async_copy(x_hbm.at[pl.ds(off, _ROWS)], xbufs[b], isems[b])

        def wait_in(b):
            pltpu.make_async_copy(
                x_hbm.at[pl.ds(0, _ROWS)], xbufs[b], isems[b]).wait()

        def start_out(k, b):
            off = out_base + (widl + nw * k) * _ROWS
            pltpu.async_copy(obufs[b], out_hbm.at[pl.ds(off, _ROWS)],
                             osems[b])

        def wait_out(b):
            pltpu.make_async_copy(
                obufs[b], out_hbm.at[pl.ds(0, _ROWS)], osems[b]).wait()

        start_in(0, 0)
        start_in(1, 1)

        @pl.loop(0, maxk - 2, step=2)
        def _pair(k):
            for b in range(2):
                kk = k + b
                wait_in(b)

                @pl.when(kk >= 2)
                def _drain():
                    wait_out(b)

                compute(b)
                start_out(kk, b)

                @pl.when(kk + 2 < nk)
                def _next():
                    start_in(kk + 2, b)

        wait_in(0)
        wait_out(0)
        compute(0)
        start_out(maxk - 2, 0)

        @pl.when(nk == maxk)
        def _tail():
            wait_in(1)
            wait_out(1)
            compute(1)
            start_out(maxk - 1, 1)

        wait_out(0)
        wait_out(1)

    rows1 = x1_hbm.shape[0]

    @pl.when(wid < _NW // 2)
    def _grp1():
        run_pipeline(x1_hbm, wid, _NW // 2, base0)

    @pl.when(wid >= _NW // 2)
    def _grp2():
        run_pipeline(x2_hbm, wid - _NW // 2, _NW // 2, base0 + rows1)

    if assemble:
        achunks = prev_hbm.shape[0] // _ROWS
        amax = -(-achunks // _NW)
        ank = jnp.where(wid < achunks % _NW, amax, amax - 1)

        def acopy(k):
            off = (wid + _NW * k) * _ROWS
            pltpu.sync_copy(prev_hbm.at[pl.ds(off, _ROWS)], xb0)
            pltpu.sync_copy(xb0, out_hbm.at[pl.ds(off, _ROWS)])

        @pl.loop(0, amax - 1)
        def _acopy(k):
            acopy(k)

        @pl.when(ank == amax)
        def _alast():
            acopy(amax - 1)


def _make_sc_kernel(rows, assemble=False):
    import functools
    mesh = plsc.VectorSubcoreMesh(core_axis_name="c", subcore_axis_name="s")
    return pl.kernel(
        functools.partial(_sc_body, assemble=assemble),
        mesh=mesh,
        compiler_params=pltpu.CompilerParams(
            needs_layout_passes=False, use_tc_tiling_on_sc=False),
        out_type=jax.ShapeDtypeStruct((rows, _W), jnp.float32),
        scratch_types=[
            pltpu.VMEM((_ROWS, _W), jnp.float32),
            pltpu.VMEM((_ROWS, _W), jnp.float32),
            pltpu.VMEM((_ROWS, _W), jnp.float32),
            pltpu.VMEM((_ROWS, _W), jnp.float32),
            pltpu.VMEM((_PHI_PAD,), jnp.float32),
            pltpu.VMEM((_PHI2_PAD,), jnp.float32),
            pltpu.VMEM((32,), jnp.float32),
            pltpu.VMEM((_F1PAD,), jnp.float32),
            pltpu.VMEM((_F2PAD,), jnp.float32),
            pltpu.SemaphoreType.DMA,
            pltpu.SemaphoreType.DMA,
            pltpu.SemaphoreType.DMA,
            pltpu.SemaphoreType.DMA,
        ],
    )


def kernel(x, phi_coeffs, Phi_coeffs, lambdas, eta):
    n = x.shape[0]
    rows = n // _W
    rows_a = rows // 2
    na = rows_a * _W
    phi_p = jnp.zeros((_PHI_PAD,), jnp.float32).at[:_PHI_N].set(phi_coeffs)
    big_p = jnp.zeros((_PHI2_PAD,), jnp.float32).at[:_PHI2_N].set(Phi_coeffs)
    par = jnp.concatenate([
        jnp.full((16,), eta, jnp.float32),
        jnp.full((16,), lambdas[0], jnp.float32),
    ])
    rows_1 = (rows_a // (2 * _ROWS)) * _ROWS
    n1 = rows_1 * _W

    def slc(lo, hi):
        return x[lo:hi].reshape((hi - lo) // _W, _W)

    out_a = _make_sc_kernel(rows_a)(
        slc(0, n1), slc(n1, na), phi_p, big_p, par)
    out = _make_sc_kernel(rows, assemble=True)(
        slc(na, na + n1), slc(na + n1, n), phi_p, big_p, par, out_a)
    return out.reshape(n, 1)


# speedup vs baseline: 1.0377x; 1.0377x over previous
import jax
import jax.numpy as jnp
from jax import lax
from jax.experimental import pallas as pl
from jax.experimental.pallas import tpu as pltpu
from jax.experimental.pallas import tpu_sc as plsc

_NW = 32
_W = 128
_ROWS = 125
_VPR = _W // 16
_MAXK = 8

_PHI_N = 200
_PHI2_N = 100
_PHI_PAD = 208
_PHI2_PAD = 112
_HIDDEN = 3
_F1G = _PHI_N * 32 - 32
_F2G = (_PHI2_N - 1) * 64
_F1PAD = 6384
_F2PAD = 6352


def _sc_body(*refs, assemble):
    if assemble:
        (x_hbm, phi_hbm, big_hbm, par_hbm, prev_hbm, out_hbm,
         xb0, xb1, ob0, ob1, phib, bigb, parb, f1b, f2b,
         isem0, isem1, osem0, osem1) = refs
        out_base = prev_hbm.shape[0]
    else:
        (x_hbm, phi_hbm, big_hbm, par_hbm, out_hbm,
         xb0, xb1, ob0, ob1, phib, bigb, parb, f1b, f2b,
         isem0, isem1, osem0, osem1) = refs
        out_base = 0
    nchunks = x_hbm.shape[0] // _ROWS
    maxk = -(-nchunks // _NW)
    assert maxk % 2 == 0 and maxk >= 4
    wid = lax.axis_index("s") * 2 + lax.axis_index("c")
    nk = jnp.where(wid < nchunks % _NW, maxk, maxk - 1)

    pltpu.sync_copy(phi_hbm, phib)
    pltpu.sync_copy(big_hbm, bigb)
    pltpu.sync_copy(par_hbm, parb)
    eta_v = parb[pl.ds(0, 16)]
    lam_v = parb[pl.ds(16, 16)]
    shift = [eta_v * (float(_F1G) * q) + 0.5 for q in range(_HIDDEN)]
    lam2 = lam_v * (float(_F2G) / 6.0)
    cst2 = [(q + 3.0) * (float(_F2G) / 6.0) + 0.5 for q in range(_HIDDEN)]

    iota = lax.iota(jnp.int32, 16)

    @plsc.parallel_loop(0, _F1PAD // 16)
    def _build1(j):
        g = iota + j * 16
        u = jnp.minimum(g.astype(jnp.float32) * (1.0 / 32.0),
                        float(_PHI_N - 1))
        ii = u.astype(jnp.int32)
        t = u - ii.astype(jnp.float32)
        c0 = plsc.load_gather(phib, [ii])
        c1 = plsc.load_gather(phib, [ii + 1])
        f1b[pl.ds(j * 16, 16)] = c0 + t * (c1 - c0)

    @plsc.parallel_loop(0, _F2PAD // 16)
    def _build2(j):
        g = iota + j * 16
        u = jnp.minimum(g.astype(jnp.float32) * (1.0 / 64.0),
                        float(_PHI2_N - 1))
        ii = u.astype(jnp.int32)
        t = u - ii.astype(jnp.float32)
        d0 = plsc.load_gather(bigb, [ii])
        d1 = plsc.load_gather(bigb, [ii + 1])
        f2b[pl.ds(j * 16, 16)] = d0 + t * (d1 - d0)

    xbufs, obufs = (xb0, xb1), (ob0, ob1)
    isems, osems = (isem0, isem1), (osem0, osem1)

    def start_in(k, b):
        off = (wid + _NW * k) * _ROWS
        pltpu.async_copy(x_hbm.at[pl.ds(off, _ROWS)], xbufs[b], isems[b])

    def wait_in(b):
        pltpu.make_async_copy(
            x_hbm.at[pl.ds(0, _ROWS)], xbufs[b], isems[b]).wait()

    def start_out(k, b):
        off = out_base + (wid + _NW * k) * _ROWS
        pltpu.async_copy(obufs[b], out_hbm.at[pl.ds(off, _ROWS)], osems[b])

    def wait_out(b):
        pltpu.make_async_copy(
            obufs[b], out_hbm.at[pl.ds(0, _ROWS)], osems[b]).wait()

    def compute(b):
        xb, ob = xbufs[b], obufs[b]

        @plsc.parallel_loop(0, _ROWS)
        def _row(r):
            for c in range(_VPR):
                v = xb[r, pl.ds(c * 16, 16)]
                acc = None
                for q in range(_HIDDEN):
                    g1 = jnp.minimum(v * float(_F1G) + shift[q], _F1G + 0.49)
                    phi = plsc.load_gather(f1b, [g1.astype(jnp.int32)])
                    g2 = jnp.clip(phi * lam2 + cst2[q], 0.0, _F2G + 0.49)
                    r_ = plsc.load_gather(f2b, [g2.astype(jnp.int32)])
                    acc = r_ if acc is None else acc + r_
                ob[r, pl.ds(c * 16, 16)] = acc

    start_in(0, 0)
    start_in(1, 1)

    @pl.loop(0, maxk - 2, step=2)
    def _pair(k):
        for b in range(2):
            kk = k + b
            wait_in(b)

            @pl.when(kk >= 2)
            def _drain():
                wait_out(b)

            compute(b)
            start_out(kk, b)

            @pl.when(kk + 2 < nk)
            def _next():
                start_in(kk + 2, b)

    wait_in(0)
    wait_out(0)
    compute(0)
    start_out(maxk - 2, 0)

    @pl.when(nk == maxk)
    def _tail():
        wait_in(1)
        wait_out(1)
        compute(1)
        start_out(maxk - 1, 1)

    wait_out(0)
    wait_out(1)

    if assemble:
        achunks = prev_hbm.shape[0] // _ROWS
        amax = -(-achunks // _NW)
        ank = jnp.where(wid < achunks % _NW, amax, amax - 1)

        def acopy(k):
            off = (wid + _NW * k) * _ROWS
            pltpu.sync_copy(prev_hbm.at[pl.ds(off, _ROWS)], xb0)
            pltpu.sync_copy(xb0, out_hbm.at[pl.ds(off, _ROWS)])

        @pl.loop(0, amax - 1)
        def _acopy(k):
            acopy(k)

        @pl.when(ank == amax)
        def _alast():
            acopy(amax - 1)


def _make_sc_kernel(rows, assemble=False):
    import functools
    mesh = plsc.VectorSubcoreMesh(core_axis_name="c", subcore_axis_name="s")
    return pl.kernel(
        functools.partial(_sc_body, assemble=assemble),
        mesh=mesh,
        compiler_params=pltpu.CompilerParams(
            needs_layout_passes=False, use_tc_tiling_on_sc=False),
        out_type=jax.ShapeDtypeStruct((rows, _W), jnp.float32),
        scratch_types=[
            pltpu.VMEM((_ROWS, _W), jnp.float32),
            pltpu.VMEM((_ROWS, _W), jnp.float32),
            pltpu.VMEM((_ROWS, _W), jnp.float32),
            pltpu.VMEM((_ROWS, _W), jnp.float32),
            pltpu.VMEM((_PHI_PAD,), jnp.float32),
            pltpu.VMEM((_PHI2_PAD,), jnp.float32),
            pltpu.VMEM((32,), jnp.float32),
            pltpu.VMEM((_F1PAD,), jnp.float32),
            pltpu.VMEM((_F2PAD,), jnp.float32),
            pltpu.SemaphoreType.DMA,
            pltpu.SemaphoreType.DMA,
            pltpu.SemaphoreType.DMA,
            pltpu.SemaphoreType.DMA,
        ],
    )


def kernel(x, phi_coeffs, Phi_coeffs, lambdas, eta):
    n = x.shape[0]
    rows = n // _W
    rows_a = rows // 2
    na = rows_a * _W
    phi_p = jnp.zeros((_PHI_PAD,), jnp.float32).at[:_PHI_N].set(phi_coeffs)
    big_p = jnp.zeros((_PHI2_PAD,), jnp.float32).at[:_PHI2_N].set(Phi_coeffs)
    par = jnp.concatenate([
        jnp.full((16,), eta, jnp.float32),
        jnp.full((16,), lambdas[0], jnp.float32),
    ])
    xa = x[:na].reshape(rows_a, _W)
    xb = x[na:].reshape(rows - rows_a, _W)
    out_a = _make_sc_kernel(rows_a)(xa, phi_p, big_p, par)
    out = _make_sc_kernel(rows, assemble=True)(xb, phi_p, big_p, par, out_a)
    return out.reshape(n, 1)
